# Initial kernel scaffold; baseline (speedup 1.0000x reference)
import jax
import jax.numpy as jnp
from jax.experimental import pallas as pl


def kernel(head_indices, tail_indices, node_embedding, relation_vector):
    raise NotImplementedError("write your pallas kernel here")


# speedup vs baseline: 1.0712x; 1.0712x over previous
import functools

import jax
import jax.numpy as jnp
from jax import lax
from jax.experimental import pallas as pl
from jax.experimental.pallas import tpu as pltpu
from jax.experimental.pallas import tpu_sc as plsc

N_NODES = 100000
EMBED_DIM = 128
BATCH = 16384

L = 16
NUM_CORES = 2
NUM_SUBCORES = 16
NW = NUM_CORES * NUM_SUBCORES
B_PER_W = BATCH // NW
CHUNK = 128
N_CHUNKS = B_PER_W // CHUNK
N_SEG = EMBED_DIM // L
TR_STRIDE = L + 1

_mesh = plsc.VectorSubcoreMesh(core_axis_name="c", subcore_axis_name="s")


@functools.partial(
    pl.kernel,
    mesh=_mesh,
    out_type=jax.ShapeDtypeStruct((BATCH,), jnp.float32),
    scratch_types=[
        pltpu.VMEM((B_PER_W,), jnp.int32),
        pltpu.VMEM((B_PER_W,), jnp.int32),
        pltpu.VMEM((CHUNK, EMBED_DIM), jnp.float32),
        pltpu.VMEM((CHUNK, EMBED_DIM), jnp.float32),
        pltpu.VMEM((EMBED_DIM,), jnp.float32),
        pltpu.VMEM((B_PER_W,), jnp.float32),
        pltpu.VMEM((L * TR_STRIDE,), jnp.float32),
        pltpu.SemaphoreType.DMA,
        pltpu.SemaphoreType.DMA,
    ],
    compiler_params=pltpu.CompilerParams(needs_layout_passes=False),
)
def _distmult_sc(head_hbm, tail_hbm, table_hbm, rel_hbm, out_hbm,
                 hidx_v, tidx_v, h_v, t_v, r_v, o_v, tr_v, sem_h, sem_t):
    wid = lax.axis_index("s") * NUM_CORES + lax.axis_index("c")
    base = wid * B_PER_W

    pltpu.sync_copy(head_hbm.at[pl.ds(base, B_PER_W)], hidx_v)
    pltpu.sync_copy(tail_hbm.at[pl.ds(base, B_PER_W)], tidx_v)
    pltpu.sync_copy(rel_hbm, r_v)

    tr_idx = lax.iota(jnp.int32, L) * TR_STRIDE

    for c in range(N_CHUNKS):
        cp_h = pltpu.async_copy(
            table_hbm.at[hidx_v.at[pl.ds(c * CHUNK, CHUNK)]], h_v, sem_h)
        cp_t = pltpu.async_copy(
            table_hbm.at[tidx_v.at[pl.ds(c * CHUNK, CHUNK)]], t_v, sem_t)
        cp_h.wait()
        cp_t.wait()

        def _groups(g, carry, c=c):
            for u in range(L):
                b = g * L + u
                acc = (h_v[b, pl.ds(0, L)] * t_v[b, pl.ds(0, L)]
                       * r_v[pl.ds(0, L)])
                for k in range(1, N_SEG):
                    acc = acc + (h_v[b, pl.ds(k * L, L)]
                                 * t_v[b, pl.ds(k * L, L)]
                                 * r_v[pl.ds(k * L, L)])
                plsc.store_scatter(tr_v, [tr_idx + u], acc)
            sv = tr_v[pl.ds(0, L)]
            for l in range(1, L):
                sv = sv + tr_v[pl.ds(l * TR_STRIDE, L)]
            o_v[pl.ds(c * CHUNK + g * L, L)] = sv
            return carry

        lax.fori_loop(0, CHUNK // L, _groups, 0)

    pltpu.sync_copy(o_v, out_hbm.at[pl.ds(base, B_PER_W)])


def kernel(head_indices, tail_indices, node_embedding, relation_vector):
    return _distmult_sc(head_indices, tail_indices, node_embedding,
                        relation_vector)


# speedup vs baseline: 1.4133x; 1.3194x over previous
import functools

import jax
import jax.numpy as jnp
from jax import lax
from jax.experimental import pallas as pl
from jax.experimental.pallas import tpu as pltpu
from jax.experimental.pallas import tpu_sc as plsc

N_NODES = 100000
EMBED_DIM = 128
BATCH = 16384

L = 16
NUM_CORES = 2
NUM_SUBCORES = 16
NW = NUM_CORES * NUM_SUBCORES
B_PER_W = BATCH // NW
CHUNK = 128
N_CHUNKS = B_PER_W // CHUNK
N_SEG = EMBED_DIM // L
TR_STRIDE = L + 1

_mesh = plsc.VectorSubcoreMesh(core_axis_name="c", subcore_axis_name="s")


@functools.partial(
    pl.kernel,
    mesh=_mesh,
    out_type=jax.ShapeDtypeStruct((BATCH,), jnp.float32),
    scratch_types=[
        pltpu.VMEM((B_PER_W,), jnp.int32),
        pltpu.VMEM((B_PER_W,), jnp.int32),
        pltpu.VMEM((CHUNK, EMBED_DIM), jnp.float32),
        pltpu.VMEM((CHUNK, EMBED_DIM), jnp.float32),
        pltpu.VMEM((CHUNK, EMBED_DIM), jnp.float32),
        pltpu.VMEM((CHUNK, EMBED_DIM), jnp.float32),
        pltpu.VMEM((EMBED_DIM,), jnp.float32),
        pltpu.VMEM((B_PER_W,), jnp.float32),
        pltpu.VMEM((L * TR_STRIDE,), jnp.float32),
        pltpu.SemaphoreType.DMA,
        pltpu.SemaphoreType.DMA,
        pltpu.SemaphoreType.DMA,
        pltpu.SemaphoreType.DMA,
    ],
    compiler_params=pltpu.CompilerParams(needs_layout_passes=False),
)
def _distmult_sc(head_hbm, tail_hbm, table_hbm, rel_hbm, out_hbm,
                 hidx_v, tidx_v, h0_v, t0_v, h1_v, t1_v, r_v, o_v, tr_v,
                 sem_h0, sem_t0, sem_h1, sem_t1):
    wid = lax.axis_index("s") * NUM_CORES + lax.axis_index("c")
    base = wid * B_PER_W

    cp_hi = pltpu.async_copy(head_hbm.at[pl.ds(base, B_PER_W)], hidx_v,
                             sem_h0)
    cp_ti = pltpu.async_copy(tail_hbm.at[pl.ds(base, B_PER_W)], tidx_v,
                             sem_t0)
    cp_r = pltpu.async_copy(rel_hbm, r_v, sem_h1)
    cp_hi.wait()
    cp_ti.wait()
    cp_r.wait()

    tr_idx = lax.iota(jnp.int32, L) * TR_STRIDE

    h_bufs = (h0_v, h1_v)
    t_bufs = (t0_v, t1_v)
    sems_h = (sem_h0, sem_h1)
    sems_t = (sem_t0, sem_t1)

    def _issue(c):
        p = c % 2
        cp_h = pltpu.async_copy(
            table_hbm.at[hidx_v.at[pl.ds(c * CHUNK, CHUNK)]],
            h_bufs[p], sems_h[p])
        cp_t = pltpu.async_copy(
            table_hbm.at[tidx_v.at[pl.ds(c * CHUNK, CHUNK)]],
            t_bufs[p], sems_t[p])
        return cp_h, cp_t

    pending = _issue(0)
    for c in range(N_CHUNKS):
        p = c % 2
        h_v, t_v = h_bufs[p], t_bufs[p]
        pending[0].wait()
        pending[1].wait()
        if c + 1 < N_CHUNKS:
            pending = _issue(c + 1)

        def _groups(g, carry, c=c, h_v=h_v, t_v=t_v):
            b0 = g * L
            for ub in range(0, L, 8):
                accs = [None] * 8
                for k in range(N_SEG):
                    rk = r_v[pl.ds(k * L, L)]
                    for u in range(8):
                        p_ = (h_v[b0 + ub + u, pl.ds(k * L, L)]
                              * t_v[b0 + ub + u, pl.ds(k * L, L)] * rk)
                        accs[u] = p_ if k == 0 else accs[u] + p_
                for u in range(8):
                    plsc.store_scatter(tr_v, [tr_idx + ub + u], accs[u])
            sv = tr_v[pl.ds(0, L)]
            for l in range(1, L):
                sv = sv + tr_v[pl.ds(l * TR_STRIDE, L)]
            o_v[pl.ds(c * CHUNK + g * L, L)] = sv
            return carry

        lax.fori_loop(0, CHUNK // L, _groups, 0)

    pltpu.sync_copy(o_v, out_hbm.at[pl.ds(base, B_PER_W)])


def kernel(head_indices, tail_indices, node_embedding, relation_vector):
    return _distmult_sc(head_indices, tail_indices, node_embedding,
                        relation_vector)


# speedup vs baseline: 1.4137x; 1.0003x over previous
import functools

import jax
import jax.numpy as jnp
from jax import lax
from jax.experimental import pallas as pl
from jax.experimental.pallas import tpu as pltpu
from jax.experimental.pallas import tpu_sc as plsc

N_NODES = 100000
EMBED_DIM = 128
BATCH = 16384

L = 16
NUM_CORES = 2
NUM_SUBCORES = 16
NW = NUM_CORES * NUM_SUBCORES
B_PER_W = BATCH // NW
CHUNK = 128
N_CHUNKS = B_PER_W // CHUNK
N_SEG = EMBED_DIM // L
TR_STRIDE = L + 1

_mesh = plsc.VectorSubcoreMesh(core_axis_name="c", subcore_axis_name="s")


@functools.partial(
    pl.kernel,
    mesh=_mesh,
    out_type=jax.ShapeDtypeStruct((BATCH,), jnp.float32),
    scratch_types=[
        pltpu.VMEM((B_PER_W,), jnp.int32),
        pltpu.VMEM((B_PER_W,), jnp.int32),
        pltpu.VMEM((CHUNK, EMBED_DIM), jnp.float32),
        pltpu.VMEM((CHUNK, EMBED_DIM), jnp.float32),
        pltpu.VMEM((CHUNK, EMBED_DIM), jnp.float32),
        pltpu.VMEM((CHUNK, EMBED_DIM), jnp.float32),
        pltpu.VMEM((EMBED_DIM,), jnp.float32),
        pltpu.VMEM((B_PER_W,), jnp.float32),
        pltpu.VMEM((L * TR_STRIDE,), jnp.float32),
        pltpu.SemaphoreType.DMA,
        pltpu.SemaphoreType.DMA,
        pltpu.SemaphoreType.DMA,
        pltpu.SemaphoreType.DMA,
    ],
    compiler_params=pltpu.CompilerParams(
        needs_layout_passes=False,
        skip_device_barrier=True,
        disable_bounds_checks=True,
        disable_semaphore_checks=True,
    ),
)
def _distmult_sc(head_hbm, tail_hbm, table_hbm, rel_hbm, out_hbm,
                 hidx_v, tidx_v, h0_v, t0_v, h1_v, t1_v, r_v, o_v, tr_v,
                 sem_h0, sem_t0, sem_h1, sem_t1):
    wid = lax.axis_index("s") * NUM_CORES + lax.axis_index("c")
    base = wid * B_PER_W

    cp_hi = pltpu.async_copy(head_hbm.at[pl.ds(base, B_PER_W)], hidx_v,
                             sem_h0)
    cp_ti = pltpu.async_copy(tail_hbm.at[pl.ds(base, B_PER_W)], tidx_v,
                             sem_t0)
    cp_r = pltpu.async_copy(rel_hbm, r_v, sem_h1)
    cp_hi.wait()
    cp_ti.wait()
    cp_r.wait()

    tr_idx = lax.iota(jnp.int32, L) * TR_STRIDE

    h_bufs = (h0_v, h1_v)
    t_bufs = (t0_v, t1_v)
    sems_h = (sem_h0, sem_h1)
    sems_t = (sem_t0, sem_t1)

    def _issue(c):
        p = c % 2
        cp_h = pltpu.async_copy(
            table_hbm.at[hidx_v.at[pl.ds(c * CHUNK, CHUNK)]],
            h_bufs[p], sems_h[p])
        cp_t = pltpu.async_copy(
            table_hbm.at[tidx_v.at[pl.ds(c * CHUNK, CHUNK)]],
            t_bufs[p], sems_t[p])
        return cp_h, cp_t

    pending = _issue(0)
    for c in range(N_CHUNKS):
        p = c % 2
        h_v, t_v = h_bufs[p], t_bufs[p]
        pending[0].wait()
        pending[1].wait()
        if c + 1 < N_CHUNKS:
            pending = _issue(c + 1)

        def _groups(g, carry, c=c, h_v=h_v, t_v=t_v):
            b0 = g * L
            for ub in range(0, L, 8):
                accs = [None] * 8
                for k in range(N_SEG):
                    rk = r_v[pl.ds(k * L, L)]
                    for u in range(8):
                        p_ = (h_v[b0 + ub + u, pl.ds(k * L, L)]
                              * t_v[b0 + ub + u, pl.ds(k * L, L)] * rk)
                        accs[u] = p_ if k == 0 else accs[u] + p_
                for u in range(8):
                    plsc.store_scatter(tr_v, [tr_idx + ub + u], accs[u])
            sv = tr_v[pl.ds(0, L)]
            for l in range(1, L):
                sv = sv + tr_v[pl.ds(l * TR_STRIDE, L)]
            o_v[pl.ds(c * CHUNK + g * L, L)] = sv
            return carry

        lax.fori_loop(0, CHUNK // L, _groups, 0)

    pltpu.sync_copy(o_v, out_hbm.at[pl.ds(base, B_PER_W)])


def kernel(head_indices, tail_indices, node_embedding, relation_vector):
    return _distmult_sc(head_indices, tail_indices, node_embedding,
                        relation_vector)
